# baseline (speedup 1.0000x reference)
import jax
import jax.numpy as jnp
from jax.experimental import pallas as pl


def _permute_kernel(p_ref, r_ref, po_ref, ro_ref):
    po_ref[...] = jnp.transpose(p_ref[...], (0, 2, 1))
    ro_ref[...] = jnp.transpose(r_ref[...], (0, 2, 1))


def kernel(preds, regs):
    bs, c2, fh, fw = preds.shape
    c4 = regs.shape[1]
    hw = fh * fw
    p = preds.reshape(bs, c2, hw)
    r = regs.reshape(bs, c4, hw)
    po, ro = pl.pallas_call(
        _permute_kernel,
        grid=(bs,),
        in_specs=[
            pl.BlockSpec((1, c2, hw), lambda i: (i, 0, 0)),
            pl.BlockSpec((1, c4, hw), lambda i: (i, 0, 0)),
        ],
        out_specs=[
            pl.BlockSpec((1, hw, c2), lambda i: (i, 0, 0)),
            pl.BlockSpec((1, hw, c4), lambda i: (i, 0, 0)),
        ],
        out_shape=[
            jax.ShapeDtypeStruct((bs, hw, c2), preds.dtype),
            jax.ShapeDtypeStruct((bs, hw, c4), regs.dtype),
        ],
    )(p, r)
    return po.reshape(bs, fh, fw, c2), ro.reshape(bs, fh, fw, c4 // 4, 4)


# speedup vs baseline: 1.4817x; 1.4817x over previous
import jax
import jax.numpy as jnp
from jax.experimental import pallas as pl


def _permute_kernel(p_ref, r_ref, po_ref, ro_ref):
    nb, c2, fh, fw = p_ref.shape
    c4 = r_ref.shape[1]
    tp = jnp.transpose(p_ref[...], (0, 2, 3, 1))
    tr = jnp.transpose(r_ref[...], (0, 2, 3, 1))
    po_ref[...] = tp.reshape(nb, fh, fw * c2)
    ro_ref[...] = tr.reshape(nb, fh, fw * c4)


def kernel(preds, regs):
    bs, c2, fh, fw = preds.shape
    c4 = regs.shape[1]
    po, ro = pl.pallas_call(
        _permute_kernel,
        grid=(bs,),
        in_specs=[
            pl.BlockSpec((1, c2, fh, fw), lambda i: (i, 0, 0, 0)),
            pl.BlockSpec((1, c4, fh, fw), lambda i: (i, 0, 0, 0)),
        ],
        out_specs=[
            pl.BlockSpec((1, fh, fw * c2), lambda i: (i, 0, 0)),
            pl.BlockSpec((1, fh, fw * c4), lambda i: (i, 0, 0)),
        ],
        out_shape=[
            jax.ShapeDtypeStruct((bs, fh, fw * c2), preds.dtype),
            jax.ShapeDtypeStruct((bs, fh, fw * c4), regs.dtype),
        ],
    )(preds, regs)
    return po.reshape(bs, fh, fw, c2), ro.reshape(bs, fh, fw, c4 // 4, 4)
